# baseline (speedup 1.0000x reference)
import functools

import jax
import jax.numpy as jnp
from jax import lax
from jax.experimental import pallas as pl
from jax.experimental.pallas import tpu as pltpu
from jax.experimental.pallas import tpu_sc as plsc

NUM_SPK = 100000
EMBD_DIM = 64
BATCH = 16384

_NC = 2
_NS = 16
_NW = _NC * _NS
_BPW = BATCH // _NW
_CHUNK = 128
_NCHUNK = _BPW // _CHUNK

_TCOLS = (NUM_SPK + 127) // 128
_TC_PER_W = (_TCOLS + _NW - 1) // _NW
_LAST_FULL = NUM_SPK // 128
_TAIL = NUM_SPK - _LAST_FULL * 128

_mesh = plsc.VectorSubcoreMesh(core_axis_name="c", subcore_axis_name="s")


@functools.partial(
    pl.kernel,
    mesh=_mesh,
    out_type=jax.ShapeDtypeStruct((NUM_SPK * EMBD_DIM,), jnp.float32),
    scratch_types=[
        pltpu.VMEM((EMBD_DIM, _CHUNK + 5), jnp.float32),
        pltpu.VMEM((_CHUNK * EMBD_DIM,), jnp.float32),
        pltpu.SemaphoreType.DMA,
    ],
    compiler_params=pltpu.CompilerParams(
        use_tc_tiling_on_sc=True, needs_layout_passes=False
    ),
)
def _linearize_kernel(table_t_hbm, lin_hbm, colb_v, rowb_v, sem):
    wid = lax.axis_index("s") * _NC + lax.axis_index("c")
    lanes = lax.iota(jnp.int32, 16)

    def _col_block(c, carry):
        pltpu.sync_copy(
            table_t_hbm.at[:, pl.ds(c * _CHUNK, _CHUNK)],
            colb_v.at[:, pl.ds(0, _CHUNK)],
        )
        def _sp_body(m, carry2):
            for j in range(8):
                l = m * 8 + j
                for k in range(EMBD_DIM // 16):
                    vals = plsc.load_gather(
                        colb_v, [k * 16 + lanes, jnp.full((16,), 0, jnp.int32) + l]
                    )
                    rowb_v[pl.ds(l * EMBD_DIM + k * 16, 16)] = vals
            return carry2

        lax.fori_loop(0, _CHUNK // 8, _sp_body, 0)
        @pl.when(c < _LAST_FULL)
        def _():
            pltpu.sync_copy(
                rowb_v,
                lin_hbm.at[pl.ds(c * _CHUNK * EMBD_DIM, _CHUNK * EMBD_DIM)],
            )

        @pl.when(c == _LAST_FULL)
        def _():
            pltpu.sync_copy(
                rowb_v.at[pl.ds(0, _TAIL * EMBD_DIM)],
                lin_hbm.at[pl.ds(c * _CHUNK * EMBD_DIM, _TAIL * EMBD_DIM)],
            )

        return carry

    lo = wid * _TC_PER_W
    hi = jnp.minimum(lo + _TC_PER_W, _TCOLS)
    lax.fori_loop(lo, hi, _col_block, 0)


@functools.partial(
    pl.kernel,
    mesh=_mesh,
    out_type=jax.ShapeDtypeStruct((BATCH, EMBD_DIM), jnp.float32),
    scratch_types=[
        pltpu.VMEM((_BPW,), jnp.int32),
        pltpu.VMEM((_BPW, EMBD_DIM), jnp.float32),
        pltpu.SemaphoreType.DMA,
        pltpu.SemaphoreType.DMA,
    ],
    compiler_params=pltpu.CompilerParams(use_tc_tiling_on_sc=False),
)
def _gather_kernel(idx_hbm, table_hbm, out_hbm, idx_v, rows_v, sem_g, sem_w):
    wid = lax.axis_index("s") * _NC + lax.axis_index("c")
    base = wid * _BPW
    pltpu.sync_copy(idx_hbm.at[pl.ds(base, _BPW)], idx_v)
    gathers = [
        pltpu.async_copy(
            table_hbm.at[idx_v.at[pl.ds(j * _CHUNK, _CHUNK)]],
            rows_v.at[pl.ds(j * _CHUNK, _CHUNK)],
            sem_g,
        )
        for j in range(_NCHUNK)
    ]
    writes = []
    for j in range(_NCHUNK):
        gathers[j].wait()
        writes.append(
            pltpu.async_copy(
                rows_v.at[pl.ds(j * _CHUNK, _CHUNK)],
                out_hbm.at[pl.ds(base + j * _CHUNK, _CHUNK)],
                sem_w,
            )
        )
    for w in writes:
        w.wait()


def kernel(spk_inds, table):
    lin = _linearize_kernel(table.T)
    return _gather_kernel(
        spk_inds.astype(jnp.int32), lin.reshape(NUM_SPK, EMBD_DIM)
    )


# speedup vs baseline: 3.0973x; 3.0973x over previous
import functools

import jax
import jax.numpy as jnp
from jax import lax
from jax.experimental import pallas as pl
from jax.experimental.pallas import tpu as pltpu
from jax.experimental.pallas import tpu_sc as plsc

NUM_SPK = 100000
EMBD_DIM = 64
BATCH = 16384

_NC = 2
_NS = 16
_NW = _NC * _NS
_BPW = BATCH // _NW

_mesh = plsc.VectorSubcoreMesh(core_axis_name="c", subcore_axis_name="s")


@functools.partial(
    pl.kernel,
    mesh=_mesh,
    out_type=jax.ShapeDtypeStruct((BATCH, EMBD_DIM), jnp.float32),
    scratch_types=[
        pltpu.VMEM((_BPW,), jnp.int32),
        pltpu.VMEM((_BPW, EMBD_DIM), jnp.float32),
        pltpu.SemaphoreType.DMA,
    ],
    compiler_params=pltpu.CompilerParams(
        use_tc_tiling_on_sc=True, needs_layout_passes=False
    ),
)
def _gather_kernel(idx_hbm, table_hbm, out_hbm, idx_v, rows_v, sem_g):
    wid = lax.axis_index("s") * _NC + lax.axis_index("c")
    base = wid * _BPW
    pltpu.sync_copy(idx_hbm.at[pl.ds(base, _BPW)], idx_v)

    def _row_body(g, carry):
        svec = idx_v[pl.ds(g * 16, 16)]
        for j in range(16):
            s = svec[j]
            pltpu.async_copy(
                table_hbm.at[pl.ds(s, 1)],
                rows_v.at[pl.ds(g * 16 + j, 1)],
                sem_g,
            )

        @pl.when(g >= 1)
        def _():
            for _ in range(16):
                pltpu.make_async_copy(
                    table_hbm.at[pl.ds(0, 1)], rows_v.at[pl.ds(0, 1)], sem_g
                ).wait()

        return carry

    lax.fori_loop(0, _BPW // 16, _row_body, 0)
    for _ in range(16):
        pltpu.make_async_copy(
            table_hbm.at[pl.ds(0, 1)], rows_v.at[pl.ds(0, 1)], sem_g
        ).wait()

    pltpu.sync_copy(rows_v, out_hbm.at[pl.ds(base, _BPW)])


def kernel(spk_inds, table):
    table = lax.optimization_barrier(table)
    return _gather_kernel(spk_inds.astype(jnp.int32), table)


# speedup vs baseline: 3.5034x; 1.1311x over previous
import functools

import jax
import jax.numpy as jnp
from jax import lax
from jax.experimental import pallas as pl
from jax.experimental.pallas import tpu as pltpu
from jax.experimental.pallas import tpu_sc as plsc

NUM_SPK = 100000
EMBD_DIM = 64
BATCH = 16384

_NC = 2
_NS = 16
_NW = _NC * _NS
_BPW = BATCH // _NW

_mesh = plsc.VectorSubcoreMesh(core_axis_name="c", subcore_axis_name="s")


@functools.partial(
    pl.kernel,
    mesh=_mesh,
    out_type=jax.ShapeDtypeStruct((BATCH, EMBD_DIM), jnp.float32),
    scratch_types=[
        pltpu.VMEM((_BPW,), jnp.int32),
        pltpu.VMEM((_BPW, EMBD_DIM), jnp.float32),
        pltpu.SemaphoreType.DMA,
    ],
    compiler_params=pltpu.CompilerParams(
        use_tc_tiling_on_sc=True, needs_layout_passes=False
    ),
)
def _gather_kernel(idx_hbm, table_hbm, out_hbm, idx_v, rows_v, sem_g):
    wid = lax.axis_index("s") * _NC + lax.axis_index("c")
    base = wid * _BPW
    pltpu.sync_copy(idx_hbm.at[pl.ds(base, _BPW)], idx_v)

    def _row_body(g, carry):
        svec = idx_v[pl.ds(g * 16, 16)]
        for j in range(16):
            s = svec[j]
            pltpu.async_copy(
                table_hbm.at[pl.ds(s, 1)],
                rows_v.at[pl.ds(g * 16 + j, 1)],
                sem_g,
            )
        return carry

    lax.fori_loop(0, _BPW // 16, _row_body, 0)

    def _drain_body(g, carry):
        for _ in range(16):
            pltpu.make_async_copy(
                table_hbm.at[pl.ds(0, 1)], rows_v.at[pl.ds(0, 1)], sem_g
            ).wait()
        return carry

    lax.fori_loop(0, _BPW // 16, _drain_body, 0)

    pltpu.sync_copy(rows_v, out_hbm.at[pl.ds(base, _BPW)])


def kernel(spk_inds, table):
    table = lax.optimization_barrier(table)
    return _gather_kernel(spk_inds.astype(jnp.int32), table)


# speedup vs baseline: 3.5162x; 1.0036x over previous
import functools

import jax
import jax.numpy as jnp
from jax import lax
from jax.experimental import pallas as pl
from jax.experimental.pallas import tpu as pltpu
from jax.experimental.pallas import tpu_sc as plsc

NUM_SPK = 100000
EMBD_DIM = 64
BATCH = 16384

_NC = 2
_NS = 16
_NW = _NC * _NS
_BPW = BATCH // _NW

_mesh = plsc.VectorSubcoreMesh(core_axis_name="c", subcore_axis_name="s")


@functools.partial(
    pl.kernel,
    mesh=_mesh,
    out_type=jax.ShapeDtypeStruct((BATCH, EMBD_DIM), jnp.float32),
    scratch_types=[
        pltpu.VMEM((_BPW,), jnp.int32),
        pltpu.VMEM((_BPW, EMBD_DIM), jnp.float32),
        pltpu.SemaphoreType.DMA,
    ],
    compiler_params=pltpu.CompilerParams(
        use_tc_tiling_on_sc=True, needs_layout_passes=False
    ),
)
def _gather_kernel(idx_hbm, table_hbm, out_hbm, idx_v, rows_v, sem_g):
    wid = lax.axis_index("s") * _NC + lax.axis_index("c")
    base = wid * _BPW
    pltpu.sync_copy(idx_hbm.at[pl.ds(base, _BPW)], idx_v)

    def _row_body(g, carry):
        svec = idx_v[pl.ds(g * 16, 16)]
        for j in range(16):
            s = svec[j]
            pltpu.async_copy(
                table_hbm.at[pl.ds(s, 1)],
                rows_v.at[pl.ds(g * 16 + j, 1)],
                sem_g,
            )
        return carry

    lax.fori_loop(0, _BPW // 16, _row_body, 0)

    def _drain_body(g, carry):
        pltpu.make_async_copy(
            table_hbm.at[pl.ds(0, 16)], rows_v.at[pl.ds(0, 16)], sem_g
        ).wait()
        return carry

    lax.fori_loop(0, _BPW // 16, _drain_body, 0)

    pltpu.sync_copy(rows_v, out_hbm.at[pl.ds(base, _BPW)])


def kernel(spk_inds, table):
    table = lax.optimization_barrier(table)
    return _gather_kernel(spk_inds.astype(jnp.int32), table)
